# baseline (speedup 1.0000x reference)
import functools

import jax
import jax.numpy as jnp
from jax import lax
from jax.experimental import pallas as pl
from jax.experimental.pallas import tpu as pltpu
from jax.experimental.pallas import tpu_sc as plsc

EMBED_DIM = 32
SEQ_LEN = 200
NUM_CORES = 2
NUM_SUBCORES = 16
NUM_WORKERS = NUM_CORES * NUM_SUBCORES


@functools.partial(jax.jit, static_argnums=(3, 4))
def _sc_embed(x_flat, token_table, pos_table, b_total, d):
  b_per_w = b_total // NUM_WORKERS
  seqs_per_w = b_per_w // SEQ_LEN
  mesh = plsc.VectorSubcoreMesh(core_axis_name="c", subcore_axis_name="s")

  @functools.partial(
      pl.kernel,
      mesh=mesh,
      compiler_params=pltpu.CompilerParams(use_tc_tiling_on_sc=False),
      out_type=jax.ShapeDtypeStruct((b_total, d), jnp.float32),
      scratch_types=[
          pltpu.VMEM((b_per_w,), jnp.int32),
          pltpu.VMEM((SEQ_LEN, d), jnp.float32),
          pltpu.VMEM((SEQ_LEN, d), jnp.float32),
          pltpu.SemaphoreType.DMA,
      ],
  )
  def k(idx_hbm, table_hbm, pos_hbm, out_hbm, idx_v, pos_v, rows_v, sem):
    wid = lax.axis_index("s") * NUM_CORES + lax.axis_index("c")
    base = wid * b_per_w
    pltpu.sync_copy(idx_hbm.at[pl.ds(base, b_per_w)], idx_v)
    pltpu.sync_copy(pos_hbm, pos_v)

    def seq_body(s, carry):
      row0 = pl.multiple_of(s * SEQ_LEN, 8)
      cp0 = pltpu.async_copy(
          table_hbm.at[idx_v.at[pl.ds(row0, 128)]],
          rows_v.at[pl.ds(0, 128)], sem)
      cp1 = pltpu.async_copy(
          table_hbm.at[idx_v.at[pl.ds(row0 + 128, 72)]],
          rows_v.at[pl.ds(128, 72)], sem)
      cp0.wait()
      cp1.wait()

      def add_rows(i, c):
        r = i * 8
        for u in range(8):
          for h in range(d // 16):
            sl = pl.ds(h * 16, 16)
            rows_v[r + u, sl] = rows_v[r + u, sl] + pos_v[r + u, sl]
        return c

      lax.fori_loop(0, SEQ_LEN // 8, add_rows, 0)
      pltpu.sync_copy(rows_v, out_hbm.at[pl.ds(base + row0, SEQ_LEN)])
      return carry

    lax.fori_loop(0, seqs_per_w, seq_body, 0)

  return k(x_flat, token_table, pos_table)


def kernel(x, token_table, pos_table):
  batch, seq_len = x.shape
  d = token_table.shape[1]
  x_flat = x.reshape(batch * seq_len).astype(jnp.int32)
  out = _sc_embed(x_flat, token_table, pos_table, batch * seq_len, d)
  return out.reshape(batch, seq_len, d)


# speedup vs baseline: 1.0494x; 1.0494x over previous
import functools

import jax
import jax.numpy as jnp
from jax import lax
from jax.experimental import pallas as pl
from jax.experimental.pallas import tpu as pltpu
from jax.experimental.pallas import tpu_sc as plsc

NUM_CORES = 2
NUM_SUBCORES = 16
NUM_WORKERS = NUM_CORES * NUM_SUBCORES
UNIT = 128


@functools.partial(jax.jit, static_argnums=(3, 4, 5))
def _sc_embed(x_flat, token_table, pos_table, batch, seq_len, d):
  n_tok = batch * seq_len
  n_units = n_tok // UNIT
  units_per_w = n_units // NUM_WORKERS
  tok_per_w = units_per_w * UNIT
  units_per_s = batch // UNIT
  mesh = plsc.VectorSubcoreMesh(core_axis_name="c", subcore_axis_name="s")

  @functools.partial(
      pl.kernel,
      mesh=mesh,
      compiler_params=pltpu.CompilerParams(use_tc_tiling_on_sc=False),
      out_type=jax.ShapeDtypeStruct((n_units, UNIT, d), jnp.float32),
      scratch_types=[
          pltpu.VMEM((tok_per_w,), jnp.int32),
          pltpu.VMEM((seq_len, d), jnp.float32),
          pltpu.VMEM((2, UNIT, d), jnp.float32),
          pltpu.SemaphoreType.DMA,
          pltpu.SemaphoreType.DMA,
          pltpu.SemaphoreType.DMA,
          pltpu.SemaphoreType.DMA,
      ],
  )
  def k(idx_hbm, table_hbm, pos_hbm, out_hbm, idx_v, pos_v, rows_v,
        gsem0, gsem1, osem0, osem1):
    wid = lax.axis_index("s") * NUM_CORES + lax.axis_index("c")
    g0 = wid * units_per_w
    pltpu.sync_copy(idx_hbm.at[pl.ds(g0 * UNIT, tok_per_w)], idx_v)
    pltpu.sync_copy(pos_hbm, pos_v)

    gsems = (gsem0, gsem1)
    osems = (osem0, osem1)

    def fire_gather(u, par, sem):
      return pltpu.async_copy(
          table_hbm.at[idx_v.at[pl.ds(u * UNIT, UNIT)]],
          rows_v.at[par], sem)

    fire_gather(0, 0, gsem0)

    def unit_body(u, par):
      @pl.when(u + 1 < units_per_w)
      def _():
        @pl.when(u >= 1)
        def _():
          pltpu.make_async_copy(
              rows_v.at[1 - par], out_hbm.at[g0 + u - 1],
              osems[1 - par]).wait()
        fire_gather(u + 1, 1 - par, gsems[1 - par])

      pltpu.make_async_copy(
          table_hbm.at[idx_v.at[pl.ds(u * UNIT, UNIT)]],
          rows_v.at[par], gsems[par]).wait()

      s = (g0 + u) // units_per_s
      pv = [pos_v[s, pl.ds(h * 16, 16)] for h in range(d // 16)]

      def add_rows(i, c):
        r = i * 8
        for uu in range(8):
          for h in range(d // 16):
            sl = pl.ds(h * 16, 16)
            rows_v[par, r + uu, sl] = rows_v[par, r + uu, sl] + pv[h]
        return c

      lax.fori_loop(0, UNIT // 8, add_rows, 0)
      pltpu.async_copy(rows_v.at[par], out_hbm.at[g0 + u], osems[par])

    def pair_body(p, carry):
      unit_body(p * 2, 0)
      unit_body(p * 2 + 1, 1)
      return carry

    lax.fori_loop(0, units_per_w // 2, pair_body, 0)

    for par in range(2):
      u_last = units_per_w - 2 + par
      pltpu.make_async_copy(
          rows_v.at[par], out_hbm.at[g0 + u_last], osems[par]).wait()

  return k(x_flat, token_table, pos_table)


def kernel(x, token_table, pos_table):
  batch, seq_len = x.shape
  d = token_table.shape[1]
  x_flat = x.T.reshape(batch * seq_len).astype(jnp.int32)
  out = _sc_embed(x_flat, token_table, pos_table, batch, seq_len, d)
  return out.reshape(seq_len, batch, d).transpose(1, 0, 2)
